# baseline (speedup 1.0000x reference)
import functools

import jax
import jax.numpy as jnp
from jax import lax
from jax.experimental import pallas as pl
from jax.experimental.pallas import tpu as pltpu
from jax.experimental.pallas import tpu_sc as plsc

E = 8
TOPK = 2
DM = 1024
DF = 4096
CAP = 640
NSLOT = E * CAP
NC = 2
NS = 16
NW = NC * NS
SPW = NSLOT // NW
GCH = SPW // 2
L = 16

FT = 512
CT = 128
KT = 2048


def _gate_body(x_ref, wg_ref, d1_ref, d2_ref, k1_ref, k2_ref, g1_ref, g2_ref,
               sw_ref, xmean_ref):
    xf = x_ref[...]
    T = xf.shape[0]
    logits = jnp.dot(xf, wg_ref[...], preferred_element_type=jnp.float32)
    m = jnp.max(logits, axis=-1, keepdims=True)
    ex = jnp.exp(logits - m)
    sm = ex / jnp.sum(ex, axis=-1, keepdims=True)
    iotaE = lax.broadcasted_iota(jnp.int32, (T, E), 1)
    g1v = jnp.max(sm, axis=-1, keepdims=True)
    e1 = jnp.min(jnp.where(sm >= g1v, iotaE, E), axis=-1, keepdims=True)
    sm2 = jnp.where(iotaE == e1, -1.0, sm)
    g2v = jnp.max(sm2, axis=-1, keepdims=True)
    e2 = jnp.min(jnp.where(sm2 >= g2v, iotaE, E), axis=-1, keepdims=True)
    ssum = g1v + g2v + 1e-9
    g1n = g1v / ssum
    g2n = g2v / ssum
    mask1 = (iotaE == e1).astype(jnp.float32)
    mask2 = (iotaE == e2).astype(jnp.float32)
    both = jnp.concatenate([mask1, mask2], axis=1)
    r = lax.broadcasted_iota(jnp.int32, (T, T), 0)
    c = lax.broadcasted_iota(jnp.int32, (T, T), 1)
    tri = (r >= c).astype(jnp.float32)
    cum = jnp.dot(tri, both, preferred_element_type=jnp.float32)
    cum1 = cum[:, :E]
    cum2 = cum[:, E:]
    n1 = cum1[T - 1:T, :]
    loc1 = cum1 - 1.0
    loc2 = cum2 - 1.0 + n1
    m1k = mask1 * (loc1 < CAP).astype(jnp.float32)
    m2k = mask2 * (loc2 < CAP).astype(jnp.float32)
    pos1 = jnp.sum(loc1 * m1k, axis=-1, keepdims=True).astype(jnp.int32)
    pos2 = jnp.sum(loc2 * m2k, axis=-1, keepdims=True).astype(jnp.int32)
    kept1 = jnp.sum(m1k, axis=-1, keepdims=True)
    kept2 = jnp.sum(m2k, axis=-1, keepdims=True)
    g1 = g1n * kept1
    g2 = g2n * kept2
    d1_ref[...] = e1 * CAP + pos1
    d2_ref[...] = e2 * CAP + pos2
    k1_ref[...] = kept1.astype(jnp.int32)
    k2_ref[...] = kept2.astype(jnp.int32)
    g1_ref[...] = g1
    g2_ref[...] = g2
    sw_ref[...] = jnp.sum(m1k * g1 + m2k * g2, axis=0, keepdims=True)
    xmean_ref[...] = jnp.sum(xf, axis=0, keepdims=True) * (1.0 / T)


def _gate(xf, Wg):
    T = xf.shape[0]
    f32, i32 = jnp.float32, jnp.int32
    outs = [((T, 1), i32), ((T, 1), i32), ((T, 1), i32), ((T, 1), i32),
            ((T, 1), f32), ((T, 1), f32), ((1, E), f32), ((1, DM), f32)]
    return pl.pallas_call(
        _gate_body,
        out_shape=[jax.ShapeDtypeStruct(s, d) for s, d in outs],
    )(xf, Wg)


def _dispatch_body(d1_hbm, d2_hbm, k1_hbm, k2_hbm, g1_hbm, g2_hbm, xf_hbm,
                   buf_hbm, w_hbm,
                   d1v, d2v, k1v, k2v, g1v, g2v, tokv, wv, idxv, rowsv, sem):
    cid = lax.axis_index("c")
    sid = lax.axis_index("s")
    wid = sid * NC + cid
    T = d1v.shape[0]
    pltpu.sync_copy(d1_hbm, d1v)
    pltpu.sync_copy(d2_hbm, d2v)
    pltpu.sync_copy(k1_hbm, k1v)
    pltpu.sync_copy(k2_hbm, k2v)
    pltpu.sync_copy(g1_hbm, g1v)
    pltpu.sync_copy(g2_hbm, g2v)

    zi = jnp.zeros((L,), jnp.int32)
    zf = jnp.zeros((L,), jnp.float32)

    def zero_body(i, _):
        sl = pl.ds(i * L, L)
        tokv[sl] = zi
        wv[sl] = zf
        return 0

    lax.fori_loop(0, NSLOT // L, zero_body, 0)

    def scat_body(i, _):
        sl = pl.ds(i * L, L)
        tid = lax.iota(jnp.int32, L) + i * L
        idx1 = d1v[sl]
        m1 = k1v[sl] != 0
        plsc.store_scatter(tokv, [idx1], tid, mask=m1)
        plsc.store_scatter(wv, [idx1], g1v[sl], mask=m1)
        idx2 = d2v[sl]
        m2 = k2v[sl] != 0
        plsc.store_scatter(tokv, [idx2], tid, mask=m2)
        plsc.store_scatter(wv, [idx2], g2v[sl], mask=m2)
        return 0

    lax.fori_loop(0, T // L, scat_body, 0)

    @pl.when(jnp.logical_and(cid == 0, sid == 0))
    def _():
        pltpu.sync_copy(wv, w_hbm)

    base = wid * SPW
    for r in range(SPW // GCH):
        for j in range(GCH // L):
            idxv[pl.ds(j * L, L)] = tokv[pl.ds(base + r * GCH + j * L, L)]
        pltpu.async_copy(xf_hbm.at[idxv], rowsv, sem).wait()
        pltpu.sync_copy(rowsv, buf_hbm.at[pl.ds(base + r * GCH, GCH)])


def _dispatch(d1, d2, k1, k2, g1, g2, xf):
    T = xf.shape[0]
    mesh = plsc.VectorSubcoreMesh(core_axis_name="c", subcore_axis_name="s")
    f32, i32 = jnp.float32, jnp.int32
    kern = pl.kernel(
        _dispatch_body,
        out_type=[jax.ShapeDtypeStruct((NSLOT, DM), f32),
                  jax.ShapeDtypeStruct((NSLOT,), f32)],
        mesh=mesh,
        scratch_types=[
            pltpu.VMEM((T,), i32), pltpu.VMEM((T,), i32),
            pltpu.VMEM((T,), i32), pltpu.VMEM((T,), i32),
            pltpu.VMEM((T,), f32), pltpu.VMEM((T,), f32),
            pltpu.VMEM((NSLOT,), i32), pltpu.VMEM((NSLOT,), f32),
            pltpu.VMEM((GCH,), i32), pltpu.VMEM((GCH, DM), f32),
            pltpu.SemaphoreType.DMA,
        ],
        compiler_params=pltpu.CompilerParams(needs_layout_passes=False),
    )
    return kern(d1, d2, k1, k2, g1, g2, xf)


def _ffn1_body(buf_ref, w1_ref, b1_ref, wT_ref, hw_ref):
    e = pl.program_id(0)
    ft = pl.program_id(1)
    ct = pl.program_id(2)

    @pl.when(jnp.logical_and(ft == 0, ct == 0))
    def _():
        hw_ref[...] = jnp.zeros_like(hw_ref)

    x = buf_ref[0].astype(jnp.bfloat16)
    wm = w1_ref[0].astype(jnp.bfloat16)
    h = jnp.dot(x, wm, preferred_element_type=jnp.float32)
    b1all = b1_ref[:, pl.ds(ft * FT, FT)]
    row = lax.broadcasted_iota(jnp.int32, (E, FT), 0)
    b1row = jnp.sum(jnp.where(row == e, b1all, 0.0), axis=0, keepdims=True)
    h = jnp.maximum(h + b1row, 0.0)
    wall = wT_ref[pl.ds(ct * CT, CT), :]
    lane = lax.broadcasted_iota(jnp.int32, (CT, E), 1)
    wv = jnp.sum(jnp.where(lane == e, wall, 0.0), axis=1, keepdims=True)
    red = jnp.sum(h * wv, axis=0, keepdims=True)
    hw_ref[0, pl.ds(ft, 1), :] += red


def _ffn1(buf3, W1, b1, wT):
    grid = (E, DF // FT, CAP // CT)
    return pl.pallas_call(
        _ffn1_body,
        grid=grid,
        in_specs=[
            pl.BlockSpec((1, CT, DM), lambda e, f, c: (e, c, 0)),
            pl.BlockSpec((1, DM, FT), lambda e, f, c: (e, 0, f)),
            pl.BlockSpec((E, DF), lambda e, f, c: (0, 0)),
            pl.BlockSpec((CAP, E), lambda e, f, c: (0, 0)),
        ],
        out_specs=pl.BlockSpec((1, DF // FT, FT), lambda e, f, c: (e, 0, 0)),
        out_shape=jax.ShapeDtypeStruct((E, DF // FT, FT), jnp.float32),
    )(buf3, W1, b1, wT)


def _ffn2_body(hw_ref, w2_ref, xmean_ref, sw_ref, b2_ref, y_ref, out_ref,
               acc_ref):
    k = pl.program_id(0)
    nk = pl.num_programs(0)

    @pl.when(k == 0)
    def _():
        acc_ref[...] = jnp.zeros_like(acc_ref)

    acc_ref[...] += jnp.dot(hw_ref[...], w2_ref[...],
                            preferred_element_type=jnp.float32)

    @pl.when(k == nk - 1)
    def _():
        T = 2048.0
        bias = jnp.dot(sw_ref[...], b2_ref[...],
                       preferred_element_type=jnp.float32)
        sent = xmean_ref[...] + (acc_ref[...] + bias) * (1.0 / T)
        mm = jnp.max(sent)
        lse = jnp.log(jnp.sum(jnp.exp(sent - mm))) + mm
        col = lax.broadcasted_iota(jnp.int32, (1, DM), 1)
        tgt = jnp.sum(jnp.where(col == y_ref[0, 0], sent, 0.0))
        out_ref[0, 0] = lse - tgt


def _ffn2_loss(hwf, W2f, xmean, sw, b2, y2):
    nk = hwf.shape[1] // KT
    return pl.pallas_call(
        _ffn2_body,
        grid=(nk,),
        in_specs=[
            pl.BlockSpec((1, KT), lambda k: (0, k)),
            pl.BlockSpec((KT, DM), lambda k: (k, 0)),
            pl.BlockSpec((1, DM), lambda k: (0, 0)),
            pl.BlockSpec((1, E), lambda k: (0, 0)),
            pl.BlockSpec((E, DM), lambda k: (0, 0)),
            pl.BlockSpec(memory_space=pltpu.SMEM),
        ],
        out_specs=pl.BlockSpec(memory_space=pltpu.SMEM),
        out_shape=jax.ShapeDtypeStruct((1, 1), jnp.float32),
        scratch_shapes=[pltpu.VMEM((1, DM), jnp.float32)],
    )(hwf, W2f, xmean, sw, b2, y2)


def kernel(x, y, Wg, W1, b1, W2, b2):
    B, S, _ = x.shape
    T = B * S
    xf = x.reshape(T, DM)
    d1, d2, k1, k2, g1, g2, sw, xmean = _gate(xf, Wg)
    buf, w = _dispatch(d1.reshape(T), d2.reshape(T), k1.reshape(T),
                       k2.reshape(T), g1.reshape(T), g2.reshape(T), xf)
    buf3 = buf.reshape(E, CAP, DM)
    wT = w.reshape(E, CAP).T
    hw = _ffn1(buf3, W1, b1, wT)
    y2 = y.astype(jnp.int32).reshape(1, 1)
    loss = _ffn2_loss(hw.reshape(1, E * DF), W2.reshape(E * DF, DM),
                      xmean, sw, b2, y2)
    return loss.reshape(())
